# baseline (speedup 1.0000x reference)
import functools

import jax
import jax.numpy as jnp
from jax import lax
from jax.experimental import pallas as pl
from jax.experimental.pallas import tpu as pltpu
from jax.experimental.pallas import tpu_sc as plsc

_E = 64
_K = 8
_COEF = 0.01
_NW = 32


def _gate_body(x_ref, w_ref, b_ref, sc_ref, p_ref):
    i = pl.program_id(0)
    x = x_ref[...]
    w = w_ref[...]
    logits = jax.lax.dot_general(x, w, (((1,), (1,)), ((), ())),
                                 preferred_element_type=jnp.float32)
    logits = logits + b_ref[...]
    m = jnp.max(logits, axis=-1, keepdims=True)
    ex = jnp.exp(logits - m)
    scores = ex / jnp.sum(ex, axis=-1, keepdims=True)
    sc_ref[...] = scores

    @pl.when(i == 0)
    def _init():
        p_ref[...] = jnp.zeros_like(p_ref)

    p_ref[...] += jnp.sum(scores, axis=0, keepdims=True)


def _loss_body(cnt_ref, p_ref, loss_ref, *, n_tokens):
    cnt = jnp.sum(cnt_ref[...], axis=0, keepdims=True)
    p_i = p_ref[...] / n_tokens
    f_i = cnt / (n_tokens * _K)
    loss_ref[0, 0] = _COEF * _E * jnp.sum(f_i * p_i)


def _route_body(sc_hbm, rw_hbm, se_hbm, cnt_hbm, st_v, rw_v, se_v, cnt_v,
                *, chunk):
    wid = lax.axis_index("s") * 2 + lax.axis_index("c")
    pltpu.sync_copy(sc_hbm.at[wid], st_v)

    lane = lax.broadcasted_iota(jnp.int32, (16,), 0)
    zeros16 = jnp.zeros((16,), jnp.float32)
    for z in range(16 * _E // 16):
        cnt_v[pl.ds(z * 16, 16)] = zeros16

    def group(g, carry):
        tok = g * 16 + lane
        tokE = tok * _E
        vals, idxs = [], []
        for _k in range(_K):
            m = jnp.full((16,), -1.0, jnp.float32)
            ix = jnp.zeros((16,), jnp.int32)
            for e in range(_E):
                v = plsc.load_gather(st_v, [tokE + e])
                gt = v > m
                m = jnp.where(gt, v, m)
                ix = jnp.where(gt, jnp.full((16,), e, jnp.int32), ix)
            vals.append(m)
            idxs.append(ix)
            plsc.store_scatter(st_v, [tokE + ix],
                               jnp.full((16,), -1.0, jnp.float32))
        mm = vals[0]
        for k in range(1, _K):
            mm = jnp.maximum(mm, vals[k])
        exps = [jnp.exp(v - mm) for v in vals]
        tot = exps[0]
        for k in range(1, _K):
            tot = tot + exps[k]
        tokK = tok * _K
        for k in range(_K):
            plsc.store_scatter(rw_v, [tokK + k], exps[k] / tot)
            plsc.store_scatter(se_v, [tokK + k], idxs[k])
            plsc.addupdate_scatter(cnt_v, [lane * _E + idxs[k]],
                                   jnp.ones((16,), jnp.float32))
        return carry

    lax.fori_loop(0, chunk // 16, group, 0)

    pltpu.sync_copy(rw_v, rw_hbm.at[pl.ds(wid * chunk * _K, chunk * _K)])
    pltpu.sync_copy(se_v, se_hbm.at[pl.ds(wid * chunk * _K, chunk * _K)])
    pltpu.sync_copy(cnt_v, cnt_hbm.at[wid])


def kernel(hidden_states, W, b):
    B, S, H = hidden_states.shape
    N = B * S
    x = hidden_states.reshape(N, H)
    T = min(1024, N)
    chunk = N // _NW

    scores, p_sum = pl.pallas_call(
        _gate_body,
        grid=(N // T,),
        in_specs=[
            pl.BlockSpec((T, H), lambda i: (i, 0)),
            pl.BlockSpec((_E, H), lambda i: (0, 0)),
            pl.BlockSpec((1, _E), lambda i: (0, 0)),
        ],
        out_specs=[
            pl.BlockSpec((T, _E), lambda i: (i, 0)),
            pl.BlockSpec((1, _E), lambda i: (0, 0)),
        ],
        out_shape=[
            jax.ShapeDtypeStruct((N, _E), jnp.float32),
            jax.ShapeDtypeStruct((1, _E), jnp.float32),
        ],
    )(x, W, b.reshape(1, _E))

    mesh = plsc.VectorSubcoreMesh(core_axis_name="c", subcore_axis_name="s")
    rw, se, cnt = pl.kernel(
        functools.partial(_route_body, chunk=chunk),
        mesh=mesh,
        compiler_params=pltpu.CompilerParams(needs_layout_passes=False),
        out_type=[
            jax.ShapeDtypeStruct((N * _K,), jnp.float32),
            jax.ShapeDtypeStruct((N * _K,), jnp.int32),
            jax.ShapeDtypeStruct((_NW, 16 * _E), jnp.float32),
        ],
        scratch_types=[
            pltpu.VMEM((chunk * _E,), jnp.float32),
            pltpu.VMEM((chunk * _K,), jnp.float32),
            pltpu.VMEM((chunk * _K,), jnp.int32),
            pltpu.VMEM((16 * _E,), jnp.float32),
        ],
    )(scores.reshape(_NW, chunk * _E))

    loss = pl.pallas_call(
        functools.partial(_loss_body, n_tokens=float(N)),
        grid=(1,),
        in_specs=[
            pl.BlockSpec((_NW * 16, _E), lambda i: (0, 0)),
            pl.BlockSpec((1, _E), lambda i: (0, 0)),
        ],
        out_specs=pl.BlockSpec((1, 1), lambda i: (0, 0),
                               memory_space=pltpu.SMEM),
        out_shape=jax.ShapeDtypeStruct((1, 1), jnp.float32),
    )(cnt.reshape(_NW * 16, _E), p_sum)

    return rw.reshape(B, S, _K), se.reshape(B, S, _K), loss[0, 0]


# speedup vs baseline: 2.3689x; 2.3689x over previous
import functools

import jax
import jax.numpy as jnp
from jax.experimental import pallas as pl
from jax.experimental.pallas import tpu as pltpu

_E = 64
_K = 8
_COEF = 0.01


def _route_half(x, w, b, rw_ref, se_ref):
    logits = jax.lax.dot_general(x, w, (((1,), (1,)), ((), ())),
                                 preferred_element_type=jnp.float32)
    logits = logits + b
    m = jnp.max(logits, axis=-1, keepdims=True)
    ex = jnp.exp(logits - m)
    scores = ex / jnp.sum(ex, axis=-1, keepdims=True)

    iota = jax.lax.broadcasted_iota(jnp.int32, scores.shape, 1)
    s = scores
    vals, idxs = [], []
    for _ in range(_K):
        mk = jnp.max(s, axis=-1, keepdims=True)
        ik = jnp.argmax(s, axis=-1, keepdims=True).astype(jnp.int32)
        vals.append(mk)
        idxs.append(ik)
        s = jnp.where(iota == ik, -1.0, s)
    topv = jnp.concatenate(vals, axis=-1)
    topi = jnp.concatenate(idxs, axis=-1)

    mm = jnp.max(topv, axis=-1, keepdims=True)
    e2 = jnp.exp(topv - mm)
    rw_ref[...] = e2 / jnp.sum(e2, axis=-1, keepdims=True)
    se_ref[...] = topi

    p_part = jnp.sum(scores, axis=0, keepdims=True)
    c_part = jnp.sum((s < 0.0).astype(jnp.float32), axis=0, keepdims=True)
    return p_part, c_part


def _router_body(xa_ref, xb_ref, w_ref, b_ref, rwa_ref, sea_ref,
                 rwb_ref, seb_ref, loss_ref, acc_ref, *, n_tokens):
    i = pl.program_id(0)
    n = pl.num_programs(0)
    w = w_ref[...]
    b = b_ref[...]

    pa, ca = _route_half(xa_ref[...], w, b, rwa_ref, sea_ref)
    pb, cb = _route_half(xb_ref[...], w, b, rwb_ref, seb_ref)

    @pl.when(i == 0)
    def _init():
        acc_ref[...] = jnp.zeros_like(acc_ref)

    acc_ref[0:1, :] += pa + pb
    acc_ref[1:2, :] += ca + cb

    @pl.when(i == n - 1)
    def _fin():
        p_i = acc_ref[0:1, :] / n_tokens
        f_i = acc_ref[1:2, :] / (n_tokens * _K)
        loss_ref[0, 0] = _COEF * _E * jnp.sum(f_i * p_i)


def kernel(hidden_states, W, b):
    B, S, H = hidden_states.shape
    N = B * S
    x = hidden_states.reshape(N, H)
    TH = min(512, N // 2)
    nblk = N // (2 * TH)
    rwa, sea, rwb, seb, loss = pl.pallas_call(
        functools.partial(_router_body, n_tokens=float(N)),
        grid=(nblk,),
        in_specs=[
            pl.BlockSpec((TH, H), lambda i: (2 * i, 0)),
            pl.BlockSpec((TH, H), lambda i: (2 * i + 1, 0)),
            pl.BlockSpec((_E, H), lambda i: (0, 0)),
            pl.BlockSpec((1, _E), lambda i: (0, 0)),
        ],
        out_specs=[
            pl.BlockSpec((TH, _K), lambda i: (i, 0)),
            pl.BlockSpec((TH, _K), lambda i: (i, 0)),
            pl.BlockSpec((TH, _K), lambda i: (i, 0)),
            pl.BlockSpec((TH, _K), lambda i: (i, 0)),
            pl.BlockSpec((1, 1), lambda i: (0, 0), memory_space=pltpu.SMEM),
        ],
        out_shape=[
            jax.ShapeDtypeStruct((N // 2, _K), jnp.float32),
            jax.ShapeDtypeStruct((N // 2, _K), jnp.int32),
            jax.ShapeDtypeStruct((N // 2, _K), jnp.float32),
            jax.ShapeDtypeStruct((N // 2, _K), jnp.int32),
            jax.ShapeDtypeStruct((1, 1), jnp.float32),
        ],
        scratch_shapes=[pltpu.VMEM((2, _E), jnp.float32)],
    )(x, x, W, b.reshape(1, _E))
    rw = jnp.stack([rwa.reshape(nblk, TH, _K), rwb.reshape(nblk, TH, _K)],
                   axis=1).reshape(B, S, _K)
    se = jnp.stack([sea.reshape(nblk, TH, _K), seb.reshape(nblk, TH, _K)],
                   axis=1).reshape(B, S, _K)
    return rw, se, loss[0, 0]
